# baseline (speedup 1.0000x reference)
import functools

import jax
import jax.numpy as jnp
from jax import lax
from jax.experimental import pallas as pl
from jax.experimental.pallas import tpu as pltpu
from jax.experimental.pallas import tpu_sc as plsc

_BBLK = 128
_NW = 32
_LANES = 16


def _coef_body(x_ref, w_ref, o_ref, *, C, H, U):
    n = x_ref.shape[0]
    acc = jnp.zeros((n, U), dtype=jnp.float32)
    for c in range(C):
        for h in range(H):
            acc = acc + jnp.dot(
                x_ref[:, c, h, :], w_ref[c, h, :, :],
                preferred_element_type=jnp.float32)
    line = jax.nn.sigmoid(acc)
    l = [line[:, h:h + 1] for h in range(H)]

    def cs(a, b):
        return jnp.minimum(a, b), jnp.maximum(a, b)

    a0, a1 = cs(l[0], l[1])
    a2, a3 = cs(l[2], l[3])
    b0, b2 = cs(a0, a2)
    b1, b3 = cs(a1, a3)
    c1, c2 = cs(b1, b2)
    med = (c1 + c2) * 0.5

    m = [(med < l[h]) for h in range(H)]
    mt = [mm.astype(jnp.int32) for mm in m]
    mf = [1 - v for v in mt]
    ct, cf = [], []
    st = sf = 0
    for h in range(H):
        st = st + mt[h]
        sf = sf + mf[h]
        ct.append(st)
        cf.append(sf)
    n_t, n_f = ct[-1], cf[-1]
    rank_a = [jnp.where(m[h], ct[h] - 1, n_t + cf[h] - 1) for h in range(H)]
    rank_d = [jnp.where(m[h], n_f + ct[h] - 1, cf[h] - 1) for h in range(H)]

    for s in range(2):
        for h in range(H):
            coef = ((rank_a[h] == s).astype(jnp.float32)
                    + (rank_d[h] == s).astype(jnp.float32) * 0.0001)
            o_ref[:, s, h, :] = jnp.broadcast_to(coef, (n, _LANES))


_RSTRIP = 4
_NBUF = 2
_UNROLL = 4


def _sc_body(x_hbm, cf_hbm, o_hbm, xbuf, cbuf, obuf, *scs, B, C, H, WID):
    per = B // _NW
    steps = per // _RSTRIP
    wid = lax.axis_index("s") * 2 + lax.axis_index("c")
    base = wid * per
    sin = [scs[2 * i] for i in range(_NBUF)]
    sout = [scs[2 * i + 1] for i in range(_NBUF)]
    nlanes = 2 * H * _LANES

    pltpu.sync_copy(cf_hbm.at[pl.ds(base * nlanes, per * nlanes)], cbuf)

    def x_slice(strip):
        return x_hbm.at[pl.ds(base + strip * _RSTRIP, _RSTRIP)]

    for sl in range(_NBUF):
        pltpu.async_copy(x_slice(sl), xbuf.at[sl], sin[sl])

    def outer(g2, carry):
        for sl in range(_NBUF):
            strip = g2 * _NBUF + sl
            row = base + strip * _RSTRIP
            pltpu.make_async_copy(x_slice(strip), xbuf.at[sl],
                                  sin[sl]).wait()

            @pl.when(strip >= _NBUF)
            def _():
                pltpu.make_async_copy(
                    obuf.at[sl], o_hbm.at[pl.ds(row - _NBUF * _RSTRIP,
                                                _RSTRIP)], sout[sl]).wait()

            for r in range(_RSTRIP):
                cbase = (strip * _RSTRIP + r) * nlanes
                cfv = [[cbuf[pl.ds(cbase + (s * H + h) * _LANES, _LANES)]
                        for h in range(H)] for s in range(2)]

                def chunk(k, carry2, r=r, cfv=cfv):
                    for kk in range(_UNROLL):
                        off = (k * _UNROLL + kk) * _LANES
                        for c in range(C):
                            xv = [xbuf[sl, r, c, h, pl.ds(off, _LANES)]
                                  for h in range(H)]
                            for s in range(2):
                                acc = cfv[s][0] * xv[0]
                                for h in range(1, H):
                                    acc = acc + cfv[s][h] * xv[h]
                                obuf[sl, r, c, s, pl.ds(off, _LANES)] = acc
                    return carry2

                lax.fori_loop(0, WID // (_LANES * _UNROLL), chunk, 0)

            pltpu.async_copy(obuf.at[sl], o_hbm.at[pl.ds(row, _RSTRIP)],
                             sout[sl])

            @pl.when(strip + _NBUF < steps)
            def _():
                pltpu.async_copy(x_slice(strip + _NBUF), xbuf.at[sl],
                                 sin[sl])

        return carry

    lax.fori_loop(0, steps // _NBUF, outer, 0)

    for sl in range(_NBUF):
        row = base + (steps - _NBUF + sl) * _RSTRIP
        pltpu.make_async_copy(
            obuf.at[sl], o_hbm.at[pl.ds(row, _RSTRIP)], sout[sl]).wait()


def kernel(inputs, W):
    B, C, H, WID = inputs.shape
    U = W.shape[1]
    Wr = W.reshape(H, WID, C, U).transpose(2, 0, 1, 3)

    coef = pl.pallas_call(
        functools.partial(_coef_body, C=C, H=H, U=U),
        grid=(B // _BBLK,),
        in_specs=[
            pl.BlockSpec((_BBLK, C, H, WID), lambda i: (i, 0, 0, 0)),
            pl.BlockSpec((C, H, WID, U), lambda i: (0, 0, 0, 0)),
        ],
        out_specs=pl.BlockSpec((_BBLK, 2, H, _LANES), lambda i: (i, 0, 0, 0)),
        out_shape=jax.ShapeDtypeStruct((B, 2, H, _LANES), jnp.float32),
    )(inputs, Wr)

    sc = pl.kernel(
        functools.partial(_sc_body, B=B, C=C, H=H, WID=WID),
        mesh=plsc.VectorSubcoreMesh(core_axis_name="c", subcore_axis_name="s"),
        out_type=jax.ShapeDtypeStruct((B, C, 2, WID), jnp.float32),
        scratch_types=[
            pltpu.VMEM((_NBUF, _RSTRIP, C, H, WID), jnp.float32),
            pltpu.VMEM(((B // _NW) * 2 * H * _LANES,), jnp.float32),
            pltpu.VMEM((_NBUF, _RSTRIP, C, 2, WID), jnp.float32),
        ] + [pltpu.SemaphoreType.DMA] * (2 * _NBUF),
    )
    return sc(inputs, coef.reshape(-1))


# speedup vs baseline: 1.4589x; 1.4589x over previous
import functools

import jax
import jax.numpy as jnp
from jax import lax
from jax.experimental import pallas as pl
from jax.experimental.pallas import tpu as pltpu
from jax.experimental.pallas import tpu_sc as plsc

_BBLK = 128
_NW = 32
_LANES = 16


def _coef_body(x_ref, w_ref, o_ref, *, C, H, U):
    n = x_ref.shape[0]
    acc = jnp.zeros((n, U), dtype=jnp.float32)
    for c in range(C):
        for h in range(H):
            acc = acc + jnp.dot(
                x_ref[:, c, h, :], w_ref[c, h, :, :],
                preferred_element_type=jnp.float32)
    line = jax.nn.sigmoid(acc)
    l = [line[:, h:h + 1] for h in range(H)]

    def cs(a, b):
        return jnp.minimum(a, b), jnp.maximum(a, b)

    a0, a1 = cs(l[0], l[1])
    a2, a3 = cs(l[2], l[3])
    b0, b2 = cs(a0, a2)
    b1, b3 = cs(a1, a3)
    c1, c2 = cs(b1, b2)
    med = (c1 + c2) * 0.5

    m = [(med < l[h]) for h in range(H)]
    mt = [mm.astype(jnp.int32) for mm in m]
    mf = [1 - v for v in mt]
    ct, cf = [], []
    st = sf = 0
    for h in range(H):
        st = st + mt[h]
        sf = sf + mf[h]
        ct.append(st)
        cf.append(sf)
    n_t, n_f = ct[-1], cf[-1]
    rank_a = [jnp.where(m[h], ct[h] - 1, n_t + cf[h] - 1) for h in range(H)]
    rank_d = [jnp.where(m[h], n_f + ct[h] - 1, cf[h] - 1) for h in range(H)]

    for s in range(2):
        for h in range(H):
            coef = ((rank_a[h] == s).astype(jnp.float32)
                    + (rank_d[h] == s).astype(jnp.float32) * 0.0001)
            o_ref[:, s, h, :] = jnp.broadcast_to(coef, (n, _LANES))


_RSTRIP = 4
_NBUF = 2
_UNROLL = 4


def _sc_body(x_hbm, cf_hbm, o_hbm, xbuf, cbuf, obuf, *scs, B, C, H, WID):
    per = B // _NW
    steps = per // _RSTRIP
    wid = lax.axis_index("s") * 2 + lax.axis_index("c")
    base = wid * per
    sin = [scs[2 * i] for i in range(_NBUF)]
    sout = [scs[2 * i + 1] for i in range(_NBUF)]
    nlanes = 2 * H * _LANES

    pltpu.sync_copy(cf_hbm.at[pl.ds(base * nlanes, per * nlanes)], cbuf)

    def x_slice(strip):
        return x_hbm.at[pl.ds(base + strip * _RSTRIP, _RSTRIP)]

    for sl in range(_NBUF):
        pltpu.async_copy(x_slice(sl), xbuf.at[sl], sin[sl])

    def outer(g2, carry):
        for sl in range(_NBUF):
            strip = g2 * _NBUF + sl
            row = base + strip * _RSTRIP
            pltpu.make_async_copy(x_slice(strip), xbuf.at[sl],
                                  sin[sl]).wait()

            @pl.when(strip >= _NBUF)
            def _():
                pltpu.make_async_copy(
                    obuf.at[sl], o_hbm.at[pl.ds(row - _NBUF * _RSTRIP,
                                                _RSTRIP)], sout[sl]).wait()

            for r in range(_RSTRIP):
                cbase = (strip * _RSTRIP + r) * nlanes
                cfv = [[cbuf[pl.ds(cbase + (s * H + h) * _LANES, _LANES)]
                        for h in range(H)] for s in range(2)]

                @plsc.parallel_loop(0, WID // _LANES, unroll=_UNROLL)
                def _chunk(k, r=r, cfv=cfv, sl=sl):
                    off = k * _LANES
                    for c in range(C):
                        xv = [xbuf[sl, r, c, h, pl.ds(off, _LANES)]
                              for h in range(H)]
                        for s in range(2):
                            acc = cfv[s][0] * xv[0]
                            for h in range(1, H):
                                acc = acc + cfv[s][h] * xv[h]
                            obuf[sl, r, c, s, pl.ds(off, _LANES)] = acc

            pltpu.async_copy(obuf.at[sl], o_hbm.at[pl.ds(row, _RSTRIP)],
                             sout[sl])

            @pl.when(strip + _NBUF < steps)
            def _():
                pltpu.async_copy(x_slice(strip + _NBUF), xbuf.at[sl],
                                 sin[sl])

        return carry

    lax.fori_loop(0, steps // _NBUF, outer, 0)

    for sl in range(_NBUF):
        row = base + (steps - _NBUF + sl) * _RSTRIP
        pltpu.make_async_copy(
            obuf.at[sl], o_hbm.at[pl.ds(row, _RSTRIP)], sout[sl]).wait()


def kernel(inputs, W):
    B, C, H, WID = inputs.shape
    U = W.shape[1]
    Wr = W.reshape(H, WID, C, U).transpose(2, 0, 1, 3)

    coef = pl.pallas_call(
        functools.partial(_coef_body, C=C, H=H, U=U),
        grid=(B // _BBLK,),
        in_specs=[
            pl.BlockSpec((_BBLK, C, H, WID), lambda i: (i, 0, 0, 0)),
            pl.BlockSpec((C, H, WID, U), lambda i: (0, 0, 0, 0)),
        ],
        out_specs=pl.BlockSpec((_BBLK, 2, H, _LANES), lambda i: (i, 0, 0, 0)),
        out_shape=jax.ShapeDtypeStruct((B, 2, H, _LANES), jnp.float32),
    )(inputs, Wr)

    sc = pl.kernel(
        functools.partial(_sc_body, B=B, C=C, H=H, WID=WID),
        mesh=plsc.VectorSubcoreMesh(core_axis_name="c", subcore_axis_name="s"),
        out_type=jax.ShapeDtypeStruct((B, C, 2, WID), jnp.float32),
        scratch_types=[
            pltpu.VMEM((_NBUF, _RSTRIP, C, H, WID), jnp.float32),
            pltpu.VMEM(((B // _NW) * 2 * H * _LANES,), jnp.float32),
            pltpu.VMEM((_NBUF, _RSTRIP, C, 2, WID), jnp.float32),
        ] + [pltpu.SemaphoreType.DMA] * (2 * _NBUF),
    )
    return sc(inputs, coef.reshape(-1))


# speedup vs baseline: 1.5371x; 1.0536x over previous
import functools

import jax
import jax.numpy as jnp
from jax import lax
from jax.experimental import pallas as pl
from jax.experimental.pallas import tpu as pltpu
from jax.experimental.pallas import tpu_sc as plsc

_BBLK = 256
_NW = 32
_LANES = 16


def _coef_body(x_ref, w_ref, o_ref, *, C, H, U):
    n = x_ref.shape[0]
    acc = jnp.zeros((n, U), dtype=jnp.float32)
    for c in range(C):
        for h in range(H):
            acc = acc + jnp.dot(
                x_ref[:, c, h, :], w_ref[c, h, :, :],
                preferred_element_type=jnp.float32)
    line = jax.nn.sigmoid(acc)
    l = [line[:, h:h + 1] for h in range(H)]

    def cs(a, b):
        return jnp.minimum(a, b), jnp.maximum(a, b)

    a0, a1 = cs(l[0], l[1])
    a2, a3 = cs(l[2], l[3])
    b0, b2 = cs(a0, a2)
    b1, b3 = cs(a1, a3)
    c1, c2 = cs(b1, b2)
    med = (c1 + c2) * 0.5

    m = [(med < l[h]) for h in range(H)]
    mt = [mm.astype(jnp.int32) for mm in m]
    mf = [1 - v for v in mt]
    ct, cf = [], []
    st = sf = 0
    for h in range(H):
        st = st + mt[h]
        sf = sf + mf[h]
        ct.append(st)
        cf.append(sf)
    n_t, n_f = ct[-1], cf[-1]
    rank_a = [jnp.where(m[h], ct[h] - 1, n_t + cf[h] - 1) for h in range(H)]
    rank_d = [jnp.where(m[h], n_f + ct[h] - 1, cf[h] - 1) for h in range(H)]

    for s in range(2):
        for h in range(H):
            coef = ((rank_a[h] == s).astype(jnp.float32)
                    + (rank_d[h] == s).astype(jnp.float32) * 0.0001)
            o_ref[:, s, h, :] = jnp.broadcast_to(coef, (n, _LANES))


_RSTRIP = 4
_NBUF = 2
_UNROLL = 4


def _sc_body(x_hbm, cf_hbm, o_hbm, xbuf, cbuf, obuf, *scs, B, C, H, WID):
    per = B // _NW
    steps = per // _RSTRIP
    wid = lax.axis_index("s") * 2 + lax.axis_index("c")
    base = wid * per
    sin = [scs[2 * i] for i in range(_NBUF)]
    sout = [scs[2 * i + 1] for i in range(_NBUF)]
    nlanes = 2 * H * _LANES

    pltpu.sync_copy(cf_hbm.at[pl.ds(base * nlanes, per * nlanes)], cbuf)

    def x_slice(strip):
        return x_hbm.at[pl.ds(base + strip * _RSTRIP, _RSTRIP)]

    for sl in range(_NBUF):
        pltpu.async_copy(x_slice(sl), xbuf.at[sl], sin[sl])

    def outer(g2, carry):
        for sl in range(_NBUF):
            strip = g2 * _NBUF + sl
            row = base + strip * _RSTRIP
            pltpu.make_async_copy(x_slice(strip), xbuf.at[sl],
                                  sin[sl]).wait()

            @pl.when(strip >= _NBUF)
            def _():
                pltpu.make_async_copy(
                    obuf.at[sl], o_hbm.at[pl.ds(row - _NBUF * _RSTRIP,
                                                _RSTRIP)], sout[sl]).wait()

            for r in range(_RSTRIP):
                cbase = (strip * _RSTRIP + r) * nlanes
                cfv = [[cbuf[pl.ds(cbase + (s * H + h) * _LANES, _LANES)]
                        for h in range(H)] for s in range(2)]

                @plsc.parallel_loop(0, WID // _LANES, unroll=_UNROLL)
                def _chunk(k, r=r, cfv=cfv, sl=sl):
                    off = k * _LANES
                    for c in range(C):
                        xv = [xbuf[sl, r, c, h, pl.ds(off, _LANES)]
                              for h in range(H)]
                        for s in range(2):
                            acc = cfv[s][0] * xv[0]
                            for h in range(1, H):
                                acc = acc + cfv[s][h] * xv[h]
                            obuf[sl, r, c, s, pl.ds(off, _LANES)] = acc

            pltpu.async_copy(obuf.at[sl], o_hbm.at[pl.ds(row, _RSTRIP)],
                             sout[sl])

            @pl.when(strip + _NBUF < steps)
            def _():
                pltpu.async_copy(x_slice(strip + _NBUF), xbuf.at[sl],
                                 sin[sl])

        return carry

    lax.fori_loop(0, steps // _NBUF, outer, 0)

    for sl in range(_NBUF):
        row = base + (steps - _NBUF + sl) * _RSTRIP
        pltpu.make_async_copy(
            obuf.at[sl], o_hbm.at[pl.ds(row, _RSTRIP)], sout[sl]).wait()


def kernel(inputs, W):
    B, C, H, WID = inputs.shape
    U = W.shape[1]
    Wr = W.reshape(H, WID, C, U).transpose(2, 0, 1, 3)

    coef = pl.pallas_call(
        functools.partial(_coef_body, C=C, H=H, U=U),
        grid=(B // _BBLK,),
        in_specs=[
            pl.BlockSpec((_BBLK, C, H, WID), lambda i: (i, 0, 0, 0)),
            pl.BlockSpec((C, H, WID, U), lambda i: (0, 0, 0, 0)),
        ],
        out_specs=pl.BlockSpec((_BBLK, 2, H, _LANES), lambda i: (i, 0, 0, 0)),
        out_shape=jax.ShapeDtypeStruct((B, 2, H, _LANES), jnp.float32),
    )(inputs, Wr)

    sc = pl.kernel(
        functools.partial(_sc_body, B=B, C=C, H=H, WID=WID),
        mesh=plsc.VectorSubcoreMesh(core_axis_name="c", subcore_axis_name="s"),
        out_type=jax.ShapeDtypeStruct((B, C, 2, WID), jnp.float32),
        scratch_types=[
            pltpu.VMEM((_NBUF, _RSTRIP, C, H, WID), jnp.float32),
            pltpu.VMEM(((B // _NW) * 2 * H * _LANES,), jnp.float32),
            pltpu.VMEM((_NBUF, _RSTRIP, C, 2, WID), jnp.float32),
        ] + [pltpu.SemaphoreType.DMA] * (2 * _NBUF),
    )
    return sc(inputs, coef.reshape(-1))


# speedup vs baseline: 1.5498x; 1.0083x over previous
import functools

import jax
import jax.numpy as jnp
from jax import lax
from jax.experimental import pallas as pl
from jax.experimental.pallas import tpu as pltpu
from jax.experimental.pallas import tpu_sc as plsc

_BBLK = 512
_NW = 32
_LANES = 16


def _coef_body(x_ref, w_ref, o_ref, *, C, H, U):
    n = x_ref.shape[0]
    acc = jnp.zeros((n, U), dtype=jnp.float32)
    for c in range(C):
        for h in range(H):
            acc = acc + jnp.dot(
                x_ref[:, c, h, :], w_ref[c, h, :, :],
                preferred_element_type=jnp.float32)
    line = jax.nn.sigmoid(acc)
    l = [line[:, h:h + 1] for h in range(H)]

    def cs(a, b):
        return jnp.minimum(a, b), jnp.maximum(a, b)

    a0, a1 = cs(l[0], l[1])
    a2, a3 = cs(l[2], l[3])
    b0, b2 = cs(a0, a2)
    b1, b3 = cs(a1, a3)
    c1, c2 = cs(b1, b2)
    med = (c1 + c2) * 0.5

    m = [(med < l[h]) for h in range(H)]
    mt = [mm.astype(jnp.int32) for mm in m]
    mf = [1 - v for v in mt]
    ct, cf = [], []
    st = sf = 0
    for h in range(H):
        st = st + mt[h]
        sf = sf + mf[h]
        ct.append(st)
        cf.append(sf)
    n_t, n_f = ct[-1], cf[-1]
    rank_a = [jnp.where(m[h], ct[h] - 1, n_t + cf[h] - 1) for h in range(H)]
    rank_d = [jnp.where(m[h], n_f + ct[h] - 1, cf[h] - 1) for h in range(H)]

    for s in range(2):
        for h in range(H):
            coef = ((rank_a[h] == s).astype(jnp.float32)
                    + (rank_d[h] == s).astype(jnp.float32) * 0.0001)
            o_ref[:, s, h, :] = jnp.broadcast_to(coef, (n, _LANES))


_RSTRIP = 4
_NBUF = 2
_UNROLL = 4


def _sc_body(x_hbm, cf_hbm, o_hbm, xbuf, cbuf, obuf, *scs, B, C, H, WID):
    per = B // _NW
    steps = per // _RSTRIP
    wid = lax.axis_index("s") * 2 + lax.axis_index("c")
    base = wid * per
    sin = [scs[2 * i] for i in range(_NBUF)]
    sout = [scs[2 * i + 1] for i in range(_NBUF)]
    nlanes = 2 * H * _LANES

    pltpu.sync_copy(cf_hbm.at[pl.ds(base * nlanes, per * nlanes)], cbuf)

    def x_slice(strip):
        return x_hbm.at[pl.ds(base + strip * _RSTRIP, _RSTRIP)]

    for sl in range(_NBUF):
        pltpu.async_copy(x_slice(sl), xbuf.at[sl], sin[sl])

    def outer(g2, carry):
        for sl in range(_NBUF):
            strip = g2 * _NBUF + sl
            row = base + strip * _RSTRIP
            pltpu.make_async_copy(x_slice(strip), xbuf.at[sl],
                                  sin[sl]).wait()

            @pl.when(strip >= _NBUF)
            def _():
                pltpu.make_async_copy(
                    obuf.at[sl], o_hbm.at[pl.ds(row - _NBUF * _RSTRIP,
                                                _RSTRIP)], sout[sl]).wait()

            for r in range(_RSTRIP):
                cbase = (strip * _RSTRIP + r) * nlanes
                cfv = [[cbuf[pl.ds(cbase + (s * H + h) * _LANES, _LANES)]
                        for h in range(H)] for s in range(2)]

                @plsc.parallel_loop(0, WID // _LANES, unroll=_UNROLL)
                def _chunk(k, r=r, cfv=cfv, sl=sl):
                    off = k * _LANES
                    for c in range(C):
                        xv = [xbuf[sl, r, c, h, pl.ds(off, _LANES)]
                              for h in range(H)]
                        for s in range(2):
                            acc = cfv[s][0] * xv[0]
                            for h in range(1, H):
                                acc = acc + cfv[s][h] * xv[h]
                            obuf[sl, r, c, s, pl.ds(off, _LANES)] = acc

            pltpu.async_copy(obuf.at[sl], o_hbm.at[pl.ds(row, _RSTRIP)],
                             sout[sl])

            @pl.when(strip + _NBUF < steps)
            def _():
                pltpu.async_copy(x_slice(strip + _NBUF), xbuf.at[sl],
                                 sin[sl])

        return carry

    lax.fori_loop(0, steps // _NBUF, outer, 0)

    for sl in range(_NBUF):
        row = base + (steps - _NBUF + sl) * _RSTRIP
        pltpu.make_async_copy(
            obuf.at[sl], o_hbm.at[pl.ds(row, _RSTRIP)], sout[sl]).wait()


def kernel(inputs, W):
    B, C, H, WID = inputs.shape
    U = W.shape[1]
    Wr = W.reshape(H, WID, C, U).transpose(2, 0, 1, 3)

    coef = pl.pallas_call(
        functools.partial(_coef_body, C=C, H=H, U=U),
        grid=(B // _BBLK,),
        in_specs=[
            pl.BlockSpec((_BBLK, C, H, WID), lambda i: (i, 0, 0, 0)),
            pl.BlockSpec((C, H, WID, U), lambda i: (0, 0, 0, 0)),
        ],
        out_specs=pl.BlockSpec((_BBLK, 2, H, _LANES), lambda i: (i, 0, 0, 0)),
        out_shape=jax.ShapeDtypeStruct((B, 2, H, _LANES), jnp.float32),
    )(inputs, Wr)

    sc = pl.kernel(
        functools.partial(_sc_body, B=B, C=C, H=H, WID=WID),
        mesh=plsc.VectorSubcoreMesh(core_axis_name="c", subcore_axis_name="s"),
        out_type=jax.ShapeDtypeStruct((B, C, 2, WID), jnp.float32),
        scratch_types=[
            pltpu.VMEM((_NBUF, _RSTRIP, C, H, WID), jnp.float32),
            pltpu.VMEM(((B // _NW) * 2 * H * _LANES,), jnp.float32),
            pltpu.VMEM((_NBUF, _RSTRIP, C, 2, WID), jnp.float32),
        ] + [pltpu.SemaphoreType.DMA] * (2 * _NBUF),
    )
    return sc(inputs, coef.reshape(-1))
